# Initial kernel scaffold; baseline (speedup 1.0000x reference)
import jax
import jax.numpy as jnp
from jax.experimental import pallas as pl


def kernel(semantic_types, column_ids, is_null, is_target, is_padding, numeric_values, timestamp_values, bool_values, categorical_embed_ids, text_embed_ids, col_emb_table, cat_emb_table, text_batch_emb, W_col, b_col, W_num, b_num, W_ts, b_ts, W_cat, b_cat, W_text, b_text, bool_emb_table, identifier_emb, null_emb, mask_emb, rms_scale):
    raise NotImplementedError("write your pallas kernel here")


# speedup vs baseline: 3.9870x; 3.9870x over previous
import functools

import jax
import jax.numpy as jnp
from jax import lax
from jax.experimental import pallas as pl
from jax.experimental.pallas import tpu as pltpu
from jax.experimental.pallas import tpu_sc as plsc

B, S = 128, 200
N = B * S
D, DT = 256, 768
C, VC, U = 1000, 100000, 4096
NB, NT, TSD = 3, 7, 8
EPS = 1e-6

TOK_BLK = 256
N_BLKS = N // TOK_BLK

NC, NS = 2, 16
NW = NC * NS
PER_W = N // NW
CHUNK = 80
N_CHUNKS = PER_W // CHUNK


def _proj_body(colt_ref, textt_ref, wcol_ref, wtext_ref, bcol_ref, btext_ref,
               colp_ref, textp_ref):
    colp_ref[...] = jnp.dot(colt_ref[...].astype(jnp.bfloat16), wcol_ref[...],
                            preferred_element_type=jnp.float32) + bcol_ref[...]
    textp_ref[...] = jnp.dot(textt_ref[...].astype(jnp.bfloat16), wtext_ref[...],
                             preferred_element_type=jnp.float32) + btext_ref[...]


def _project_tables(col_emb_table, text_batch_emb, wcol_bf, wtext_bf, b_col, b_text):
    return pl.pallas_call(
        _proj_body,
        out_shape=(jax.ShapeDtypeStruct((C, D), jnp.float32),
                   jax.ShapeDtypeStruct((U, D), jnp.float32)),
    )(col_emb_table, text_batch_emb, wcol_bf, wtext_bf,
      b_col.reshape(1, D), b_text.reshape(1, D))


def _sc_gather_body(cat_tab, colp_tab, textp_tab, cat_ids, col_ids, text_ids,
                    cat_out, col_out, text_out, idx_v, cat_rows, ct_rows, sem):
    wid = lax.axis_index("s") * NC + lax.axis_index("c")
    base = wid * PER_W

    pltpu.sync_copy(cat_ids.at[pl.ds(base, PER_W)], idx_v)

    @pl.loop(0, N_CHUNKS)
    def _(ci):
        off = ci * CHUNK
        pltpu.async_copy(cat_tab.at[idx_v.at[pl.ds(off, CHUNK)]], cat_rows,
                         sem).wait()
        pltpu.sync_copy(cat_rows, cat_out.at[pl.ds(base + off, CHUNK)])

    pltpu.sync_copy(col_ids.at[pl.ds(base, PER_W)], idx_v)

    @pl.loop(0, N_CHUNKS)
    def _(ci):
        off = ci * CHUNK
        pltpu.async_copy(colp_tab.at[idx_v.at[pl.ds(off, CHUNK)]], ct_rows,
                         sem).wait()
        pltpu.sync_copy(ct_rows, col_out.at[pl.ds(base + off, CHUNK)])

    pltpu.sync_copy(text_ids.at[pl.ds(base, PER_W)], idx_v)

    @pl.loop(0, N_CHUNKS)
    def _(ci):
        off = ci * CHUNK
        pltpu.async_copy(textp_tab.at[idx_v.at[pl.ds(off, CHUNK)]], ct_rows,
                         sem).wait()
        pltpu.sync_copy(ct_rows, text_out.at[pl.ds(base + off, CHUNK)])


def _sc_gather(cat_tab, colp_tab, textp_tab, cat_ids, col_ids, text_ids):
    mesh = plsc.VectorSubcoreMesh(core_axis_name="c", subcore_axis_name="s")
    k = pl.kernel(
        _sc_gather_body,
        mesh=mesh,
        out_type=[jax.ShapeDtypeStruct((N, DT), jnp.float32),
                  jax.ShapeDtypeStruct((N, D), jnp.float32),
                  jax.ShapeDtypeStruct((N, D), jnp.float32)],
        scratch_types=[pltpu.VMEM((PER_W,), jnp.int32),
                       pltpu.VMEM((CHUNK, DT), jnp.float32),
                       pltpu.VMEM((CHUNK, D), jnp.float32),
                       pltpu.SemaphoreType.DMA],
    )
    return k(cat_tab, colp_tab, textp_tab, cat_ids, col_ids, text_ids)


def _final_body(cat_ref, colv_ref, textv_ref, stype_ref, inul_ref, itgt_ref,
                ipad_ref, num_ref, ts_ref, bool_ref, wcat_ref, bcat_ref,
                wnum_ref, bnum_ref, wts_ref, bts_ref, bemb_ref, id_ref,
                null_ref, mask_ref, scale_ref, out_ref):
    f32 = jnp.float32
    cat_val = jnp.dot(cat_ref[...].astype(jnp.bfloat16), wcat_ref[...],
                      preferred_element_type=f32) + bcat_ref[...]
    ts_val = jnp.dot(ts_ref[...].astype(jnp.bfloat16), wts_ref[...],
                     preferred_element_type=f32) + bts_ref[...]
    num_in = num_ref[...].astype(jnp.bfloat16).astype(f32)
    num_val = num_in * wnum_ref[...] + bnum_ref[...]

    be = bemb_ref[...]
    boolc = jnp.clip(bool_ref[...], 0, NB - 1)
    bool_val = ((boolc == 0).astype(f32) * be[0:1, :]
                + (boolc == 1).astype(f32) * be[1:2, :]
                + (boolc == 2).astype(f32) * be[2:3, :])

    st = stype_ref[...]
    raw = ((st == 0).astype(f32) * id_ref[...]
           + (st == 1).astype(f32) * num_val
           + (st == 2).astype(f32) * ts_val
           + (st == 3).astype(f32) * bool_val
           + (st == 4).astype(f32) * cat_val
           + (st == 5).astype(f32) * textv_ref[...])

    inul = inul_ref[...].astype(f32)
    val = inul * null_ref[...] + (1.0 - inul) * raw
    itgt = itgt_ref[...].astype(f32)
    val = itgt * mask_ref[...] + (1.0 - itgt) * val

    x = colv_ref[...] + val
    xc = x - jnp.mean(x, axis=1, keepdims=True)
    h = xc * lax.rsqrt(jnp.mean(xc * xc, axis=1, keepdims=True) + EPS)
    h = h * scale_ref[...]
    ipad = ipad_ref[...].astype(f32)
    out_ref[...] = h * (1.0 - ipad)


def _final(cat_rows, col_rows, text_rows, stype, inul, itgt, ipad, num, ts,
           boolv, wcat_bf, b_cat, W_num, b_num, wts_bf, b_ts, bool_emb_table,
           identifier_emb, null_emb, mask_emb, rms_scale):
    tok = lambda w: pl.BlockSpec((TOK_BLK, w), lambda i: (i, 0))
    rep = lambda a, b: pl.BlockSpec((a, b), lambda i: (0, 0))
    return pl.pallas_call(
        _final_body,
        grid=(N_BLKS,),
        in_specs=[tok(DT), tok(D), tok(D), tok(1), tok(1), tok(1), tok(1),
                  tok(1), tok(TSD), tok(1),
                  rep(DT, D), rep(1, D), rep(1, D), rep(1, D), rep(TSD, D),
                  rep(1, D), rep(NB, D), rep(1, D), rep(1, D), rep(1, D),
                  rep(1, D)],
        out_specs=pl.BlockSpec((TOK_BLK, D), lambda i: (i, 0)),
        out_shape=jax.ShapeDtypeStruct((N, D), jnp.float32),
    )(cat_rows, col_rows, text_rows, stype, inul, itgt, ipad, num, ts, boolv,
      wcat_bf, b_cat.reshape(1, D), W_num, b_num.reshape(1, D), wts_bf,
      b_ts.reshape(1, D), bool_emb_table, identifier_emb.reshape(1, D),
      null_emb.reshape(1, D), mask_emb.reshape(1, D), rms_scale.reshape(1, D))


def kernel(semantic_types, column_ids, is_null, is_target, is_padding,
           numeric_values, timestamp_values, bool_values,
           categorical_embed_ids, text_embed_ids,
           col_emb_table, cat_emb_table, text_batch_emb,
           W_col, b_col, W_num, b_num, W_ts, b_ts, W_cat, b_cat,
           W_text, b_text, bool_emb_table, identifier_emb, null_emb,
           mask_emb, rms_scale):
    colp, textp = _project_tables(col_emb_table, text_batch_emb,
                                  W_col.astype(jnp.bfloat16),
                                  W_text.astype(jnp.bfloat16), b_col, b_text)

    cat_ids = jnp.clip(categorical_embed_ids.astype(jnp.int32), 0, VC - 1).reshape(N)
    col_ids = jnp.clip(column_ids.astype(jnp.int32), 0, C - 1).reshape(N)
    text_ids = jnp.clip(text_embed_ids.astype(jnp.int32), 0, U - 1).reshape(N)

    cat_rows, col_rows, text_rows = _sc_gather(
        cat_emb_table, colp, textp, cat_ids, col_ids, text_ids)

    out = _final(cat_rows, col_rows, text_rows,
                 semantic_types.astype(jnp.int32).reshape(N, 1),
                 is_null.astype(jnp.int32).reshape(N, 1),
                 is_target.astype(jnp.int32).reshape(N, 1),
                 is_padding.astype(jnp.int32).reshape(N, 1),
                 numeric_values.reshape(N, 1),
                 timestamp_values.reshape(N, TSD),
                 bool_values.astype(jnp.int32).reshape(N, 1),
                 W_cat.astype(jnp.bfloat16), b_cat, W_num, b_num,
                 W_ts.astype(jnp.bfloat16), b_ts, bool_emb_table,
                 identifier_emb, null_emb, mask_emb, rms_scale)
    return out.reshape(B, S, D)


# speedup vs baseline: 4.6529x; 1.1670x over previous
import jax
import jax.numpy as jnp
from jax import lax
from jax.experimental import pallas as pl
from jax.experimental.pallas import tpu as pltpu
from jax.experimental.pallas import tpu_sc as plsc

B, S = 128, 200
N = B * S
D, DT = 256, 768
C, VC, U = 1000, 100000, 4096
NB, NT, TSD = 3, 7, 8
EPS = 1e-6

TOK_BLK = 256
N_BLKS = N // TOK_BLK

NC, NS = 2, 16
NW = NC * NS
PER_W = N // NW
CH = 40
NCH = PER_W // CH


def _proj_body(colt_ref, textt_ref, wcol_ref, wtext_ref, bcol_ref, btext_ref,
               colp_ref, textp_ref):
    colp_ref[...] = jnp.dot(colt_ref[...].astype(jnp.bfloat16), wcol_ref[...],
                            preferred_element_type=jnp.float32) + bcol_ref[...]
    textp_ref[...] = jnp.dot(textt_ref[...].astype(jnp.bfloat16), wtext_ref[...],
                             preferred_element_type=jnp.float32) + btext_ref[...]


def _project_tables(col_emb_table, text_batch_emb, wcol_bf, wtext_bf, b_col, b_text):
    return pl.pallas_call(
        _proj_body,
        out_shape=(jax.ShapeDtypeStruct((C, D), jnp.float32),
                   jax.ShapeDtypeStruct((U, D), jnp.float32)),
    )(col_emb_table, text_batch_emb, wcol_bf, wtext_bf,
      b_col.reshape(1, D), b_text.reshape(1, D))


def _pipelined_gathers(tabs, idx_refs, outs, bufs, gsem, wsem, base):
    nt = len(tabs)

    def g(t, ci, b):
        return pltpu.make_async_copy(
            tabs[t].at[idx_refs[t].at[pl.ds(ci * CH, CH)]], bufs[t][b],
            gsem.at[t, b])

    def w(t, ci, b):
        return pltpu.make_async_copy(
            bufs[t][b], outs[t].at[pl.ds(base + ci * CH, CH)], wsem.at[t, b])

    for t in range(nt):
        g(t, 0, 0).start()
    for t in range(nt):
        g(t, 0, 0).wait()
        g(t, 1, 1).start()
        w(t, 0, 0).start()

    @pl.loop(0, (NCH - 2) // 2)
    def _(k):
        for b, delta in ((1, 1), (0, 2)):
            ci = 2 * k + delta
            for t in range(nt):
                g(t, ci, b).wait()
                w(t, ci - 1, 1 - b).wait()
                g(t, ci + 1, 1 - b).start()
                w(t, ci, b).start()

    for t in range(nt):
        g(t, NCH - 1, 1).wait()
        w(t, NCH - 2, 0).wait()
        w(t, NCH - 1, 1).start()
        w(t, NCH - 1, 1).wait()


_MESH = plsc.VectorSubcoreMesh(core_axis_name="c", subcore_axis_name="s")


def _sc_cat_body(cat_tab, cat_ids, cat_out, idx_v, buf0, buf1, gsem, wsem):
    wid = lax.axis_index("s") * NC + lax.axis_index("c")
    base = wid * PER_W
    pltpu.sync_copy(cat_ids.at[pl.ds(base, PER_W)], idx_v)
    _pipelined_gathers((cat_tab,), (idx_v,), (cat_out,), ((buf0, buf1),),
                       gsem, wsem, base)


def _sc_cat_gather(cat_tab, cat_ids):
    k = pl.kernel(
        _sc_cat_body,
        mesh=_MESH,
        out_type=jax.ShapeDtypeStruct((N, DT), jnp.float32),
        scratch_types=[pltpu.VMEM((PER_W,), jnp.int32),
                       pltpu.VMEM((CH, DT), jnp.float32),
                       pltpu.VMEM((CH, DT), jnp.float32),
                       pltpu.SemaphoreType.DMA((1, 2)),
                       pltpu.SemaphoreType.DMA((1, 2))],
    )
    return k(cat_tab, cat_ids)


def _sc_proj_body(colp_tab, textp_tab, col_ids, text_ids, col_out, text_out,
                  cidx_v, tidx_v, cbuf0, cbuf1, tbuf0, tbuf1, gsem, wsem):
    wid = lax.axis_index("s") * NC + lax.axis_index("c")
    base = wid * PER_W
    pltpu.sync_copy(col_ids.at[pl.ds(base, PER_W)], cidx_v)
    pltpu.sync_copy(text_ids.at[pl.ds(base, PER_W)], tidx_v)
    _pipelined_gathers((colp_tab, textp_tab), (cidx_v, tidx_v),
                       (col_out, text_out),
                       ((cbuf0, cbuf1), (tbuf0, tbuf1)), gsem, wsem, base)


def _sc_proj_gather(colp_tab, textp_tab, col_ids, text_ids):
    k = pl.kernel(
        _sc_proj_body,
        mesh=_MESH,
        out_type=[jax.ShapeDtypeStruct((N, D), jnp.float32),
                  jax.ShapeDtypeStruct((N, D), jnp.float32)],
        scratch_types=[pltpu.VMEM((PER_W,), jnp.int32),
                       pltpu.VMEM((PER_W,), jnp.int32),
                       pltpu.VMEM((CH, D), jnp.float32),
                       pltpu.VMEM((CH, D), jnp.float32),
                       pltpu.VMEM((CH, D), jnp.float32),
                       pltpu.VMEM((CH, D), jnp.float32),
                       pltpu.SemaphoreType.DMA((2, 2)),
                       pltpu.SemaphoreType.DMA((2, 2))],
    )
    return k(colp_tab, textp_tab, col_ids, text_ids)


def _final_body(cat_ref, colv_ref, textv_ref, pk_ref, wcat_ref, bcat_ref,
                wnum_ref, bnum_ref, wts_ref, bts_ref, bemb_ref, id_ref,
                null_ref, mask_ref, scale_ref, out_ref):
    f32 = jnp.float32
    cat_val = jnp.dot(cat_ref[...].astype(jnp.bfloat16), wcat_ref[...],
                      preferred_element_type=f32) + bcat_ref[...]
    ts_val = jnp.dot(pk_ref[:, 8:16].astype(jnp.bfloat16), wts_ref[...],
                     preferred_element_type=f32) + bts_ref[...]
    num_in = pk_ref[:, 4:5].astype(jnp.bfloat16).astype(f32)
    num_val = num_in * wnum_ref[...] + bnum_ref[...]

    be = bemb_ref[...]
    boolc = jnp.clip(pk_ref[:, 5:6], 0.0, NB - 1.0)
    bool_val = ((boolc == 0.0).astype(f32) * be[0:1, :]
                + (boolc == 1.0).astype(f32) * be[1:2, :]
                + (boolc == 2.0).astype(f32) * be[2:3, :])

    st = pk_ref[:, 0:1]
    raw = ((st == 0.0).astype(f32) * id_ref[...]
           + (st == 1.0).astype(f32) * num_val
           + (st == 2.0).astype(f32) * ts_val
           + (st == 3.0).astype(f32) * bool_val
           + (st == 4.0).astype(f32) * cat_val
           + (st == 5.0).astype(f32) * textv_ref[...])

    inul = pk_ref[:, 1:2]
    val = inul * null_ref[...] + (1.0 - inul) * raw
    itgt = pk_ref[:, 2:3]
    val = itgt * mask_ref[...] + (1.0 - itgt) * val

    x = colv_ref[...] + val
    xc = x - jnp.mean(x, axis=1, keepdims=True)
    h = xc * lax.rsqrt(jnp.mean(xc * xc, axis=1, keepdims=True) + EPS)
    h = h * scale_ref[...]
    out_ref[...] = h * (1.0 - pk_ref[:, 3:4])


def _final(cat_rows, col_rows, text_rows, pk, wcat_bf, b_cat, W_num, b_num,
           wts_bf, b_ts, bool_emb_table, identifier_emb, null_emb, mask_emb,
           rms_scale):
    tok = lambda w: pl.BlockSpec((TOK_BLK, w), lambda i: (i, 0))
    rep = lambda a, b: pl.BlockSpec((a, b), lambda i: (0, 0))
    return pl.pallas_call(
        _final_body,
        grid=(N_BLKS,),
        in_specs=[tok(DT), tok(D), tok(D), tok(16),
                  rep(DT, D), rep(1, D), rep(1, D), rep(1, D), rep(TSD, D),
                  rep(1, D), rep(NB, D), rep(1, D), rep(1, D), rep(1, D),
                  rep(1, D)],
        out_specs=pl.BlockSpec((TOK_BLK, D), lambda i: (i, 0)),
        out_shape=jax.ShapeDtypeStruct((N, D), jnp.float32),
    )(cat_rows, col_rows, text_rows, pk,
      wcat_bf, b_cat.reshape(1, D), W_num, b_num.reshape(1, D), wts_bf,
      b_ts.reshape(1, D), bool_emb_table, identifier_emb.reshape(1, D),
      null_emb.reshape(1, D), mask_emb.reshape(1, D), rms_scale.reshape(1, D))


def kernel(semantic_types, column_ids, is_null, is_target, is_padding,
           numeric_values, timestamp_values, bool_values,
           categorical_embed_ids, text_embed_ids,
           col_emb_table, cat_emb_table, text_batch_emb,
           W_col, b_col, W_num, b_num, W_ts, b_ts, W_cat, b_cat,
           W_text, b_text, bool_emb_table, identifier_emb, null_emb,
           mask_emb, rms_scale):
    f32 = jnp.float32
    cat_rows = _sc_cat_gather(cat_emb_table,
                              categorical_embed_ids.astype(jnp.int32).reshape(N))

    colp, textp = _project_tables(col_emb_table, text_batch_emb,
                                  W_col.astype(jnp.bfloat16),
                                  W_text.astype(jnp.bfloat16), b_col, b_text)
    col_rows, text_rows = _sc_proj_gather(
        colp, textp, column_ids.astype(jnp.int32).reshape(N),
        text_embed_ids.astype(jnp.int32).reshape(N))

    zero = jnp.zeros((B, S), f32)
    pk = jnp.stack([semantic_types.astype(f32), is_null.astype(f32),
                    is_target.astype(f32), is_padding.astype(f32),
                    numeric_values, bool_values.astype(f32), zero, zero],
                   axis=-1)
    pk = jnp.concatenate([pk, timestamp_values], axis=-1).reshape(N, 16)

    out = _final(cat_rows, col_rows, text_rows, pk,
                 W_cat.astype(jnp.bfloat16), b_cat, W_num, b_num,
                 W_ts.astype(jnp.bfloat16), b_ts, bool_emb_table,
                 identifier_emb, null_emb, mask_emb, rms_scale)
    return out.reshape(B, S, D)


# speedup vs baseline: 6.6194x; 1.4226x over previous
import jax
import jax.numpy as jnp
from jax import lax
from jax.experimental import pallas as pl
from jax.experimental.pallas import tpu as pltpu
from jax.experimental.pallas import tpu_sc as plsc

B, S = 128, 200
N = B * S
D, DT = 256, 768
C, VC, U = 1000, 100000, 4096
NB, NT, TSD = 3, 7, 8
EPS = 1e-6

TOK_BLK = 512
N_BLKS = N // TOK_BLK
KF = 32

NC, NS = 2, 16
NW = NC * NS
PER_W = N // NW
CH = 80
NCH = PER_W // CH


def _pack_rows(p):
    pb = p.astype(jnp.bfloat16).astype(jnp.float32)
    bits = lax.bitcast_convert_type(pb, jnp.uint32)
    lo = bits[:, :D // 2] >> 16
    hi = bits[:, D // 2:] & jnp.uint32(0xFFFF0000)
    return lax.bitcast_convert_type(lo | hi, jnp.int32)


def _unpack_rows(w):
    u = lax.bitcast_convert_type(w, jnp.uint32)
    lo = lax.bitcast_convert_type(u << 16, jnp.float32)
    hi = lax.bitcast_convert_type(u & jnp.uint32(0xFFFF0000), jnp.float32)
    return jnp.concatenate([lo, hi], axis=1)


def _proj_body(colt_ref, textt_ref, wcol_ref, wtext_ref, bcol_ref, btext_ref,
               colp_ref, textp_ref):
    colp_ref[...] = _pack_rows(
        jnp.dot(colt_ref[...].astype(jnp.bfloat16), wcol_ref[...],
                preferred_element_type=jnp.float32) + bcol_ref[...])
    textp_ref[...] = _pack_rows(
        jnp.dot(textt_ref[...].astype(jnp.bfloat16), wtext_ref[...],
                preferred_element_type=jnp.float32) + btext_ref[...])


def _project_tables(col_emb_table, text_batch_emb, wcol_bf, wtext_bf, b_col, b_text):
    return pl.pallas_call(
        _proj_body,
        out_shape=(jax.ShapeDtypeStruct((C, D // 2), jnp.int32),
                   jax.ShapeDtypeStruct((U, D // 2), jnp.int32)),
    )(col_emb_table, text_batch_emb, wcol_bf, wtext_bf,
      b_col.reshape(1, D), b_text.reshape(1, D))


def _pipelined_gathers(tabs, idx_refs, outs, bufs, gsem, wsem, base):
    nt = len(tabs)

    def g(t, ci, b):
        return pltpu.make_async_copy(
            tabs[t].at[idx_refs[t].at[pl.ds(ci * CH, CH)]], bufs[t][b],
            gsem.at[t, b])

    def w(t, ci, b):
        return pltpu.make_async_copy(
            bufs[t][b], outs[t].at[pl.ds(base + ci * CH, CH)], wsem.at[t, b])

    for t in range(nt):
        g(t, 0, 0).start()
    for t in range(nt):
        g(t, 0, 0).wait()
        g(t, 1, 1).start()
        w(t, 0, 0).start()

    @pl.loop(0, (NCH - 2) // 2)
    def _(k):
        for b, delta in ((1, 1), (0, 2)):
            ci = 2 * k + delta
            for t in range(nt):
                g(t, ci, b).wait()
                w(t, ci - 1, 1 - b).wait()
                g(t, ci + 1, 1 - b).start()
                w(t, ci, b).start()

    for t in range(nt):
        g(t, NCH - 1, 1).wait()
        w(t, NCH - 2, 0).wait()
        w(t, NCH - 1, 1).start()
        w(t, NCH - 1, 1).wait()


_MESH = plsc.VectorSubcoreMesh(core_axis_name="c", subcore_axis_name="s")


def _sc_cat_body(cat_tab, cat_ids, cat_out, idx_v, buf0, buf1, gsem, wsem):
    wid = lax.axis_index("s") * NC + lax.axis_index("c")
    base = wid * PER_W
    pltpu.sync_copy(cat_ids.at[pl.ds(base, PER_W)], idx_v)
    _pipelined_gathers((cat_tab,), (idx_v,), (cat_out,), ((buf0, buf1),),
                       gsem, wsem, base)


def _sc_cat_gather(cat_tab, cat_ids):
    k = pl.kernel(
        _sc_cat_body,
        mesh=_MESH,
        out_type=jax.ShapeDtypeStruct((N, DT), jnp.float32),
        scratch_types=[pltpu.VMEM((PER_W,), jnp.int32),
                       pltpu.VMEM((CH, DT), jnp.float32),
                       pltpu.VMEM((CH, DT), jnp.float32),
                       pltpu.SemaphoreType.DMA((1, 2)),
                       pltpu.SemaphoreType.DMA((1, 2))],
    )
    return k(cat_tab, cat_ids)


def _sc_proj_body(colp_tab, textp_tab, col_ids, text_ids, col_out, text_out,
                  cidx_v, tidx_v, cbuf0, cbuf1, tbuf0, tbuf1, gsem, wsem):
    wid = lax.axis_index("s") * NC + lax.axis_index("c")
    base = wid * PER_W
    pltpu.sync_copy(col_ids.at[pl.ds(base, PER_W)], cidx_v)
    pltpu.sync_copy(text_ids.at[pl.ds(base, PER_W)], tidx_v)
    _pipelined_gathers((colp_tab, textp_tab), (cidx_v, tidx_v),
                       (col_out, text_out),
                       ((cbuf0, cbuf1), (tbuf0, tbuf1)), gsem, wsem, base)


def _sc_proj_gather(colp_tab, textp_tab, col_ids, text_ids):
    k = pl.kernel(
        _sc_proj_body,
        mesh=_MESH,
        out_type=[jax.ShapeDtypeStruct((N, D // 2), jnp.int32),
                  jax.ShapeDtypeStruct((N, D // 2), jnp.int32)],
        scratch_types=[pltpu.VMEM((PER_W,), jnp.int32),
                       pltpu.VMEM((PER_W,), jnp.int32),
                       pltpu.VMEM((CH, D // 2), jnp.int32),
                       pltpu.VMEM((CH, D // 2), jnp.int32),
                       pltpu.VMEM((CH, D // 2), jnp.int32),
                       pltpu.VMEM((CH, D // 2), jnp.int32),
                       pltpu.SemaphoreType.DMA((2, 2)),
                       pltpu.SemaphoreType.DMA((2, 2))],
    )
    return k(colp_tab, textp_tab, col_ids, text_ids)


def _final_body(cat_ref, colv_ref, textv_ref, pk_ref, wcat_ref, r_ref,
                scale_ref, out_ref):
    f32 = jnp.float32
    catd = jnp.dot(cat_ref[...].astype(jnp.bfloat16), wcat_ref[...],
                   preferred_element_type=f32)
    sel = jnp.dot(pk_ref[...], r_ref[...], preferred_element_type=f32)
    am4 = pk_ref[:, 17:18].astype(f32)
    am5 = pk_ref[:, 18:19].astype(f32)
    ipad = pk_ref[:, 19:20].astype(f32)
    x = (_unpack_rows(colv_ref[...]) + sel
         + am4 * catd + am5 * _unpack_rows(textv_ref[...]))
    xc = x - jnp.mean(x, axis=1, keepdims=True)
    h = xc * lax.rsqrt(jnp.mean(xc * xc, axis=1, keepdims=True) + EPS)
    out_ref[...] = h * scale_ref[...] * (1.0 - ipad)


def _final(cat_rows, col_rows, text_rows, pk, wcat_bf, r_mat, rms_scale):
    tok = lambda w: pl.BlockSpec((TOK_BLK, w), lambda i: (i, 0))
    rep = lambda a, b: pl.BlockSpec((a, b), lambda i: (0, 0))
    return pl.pallas_call(
        _final_body,
        grid=(N_BLKS,),
        in_specs=[tok(DT), tok(D // 2), tok(D // 2), tok(KF),
                  rep(DT, D), rep(KF, D), rep(1, D)],
        out_specs=pl.BlockSpec((TOK_BLK, D), lambda i: (i, 0)),
        out_shape=jax.ShapeDtypeStruct((N, D), jnp.float32),
    )(cat_rows, col_rows, text_rows, pk, wcat_bf, r_mat,
      rms_scale.reshape(1, D))


def kernel(semantic_types, column_ids, is_null, is_target, is_padding,
           numeric_values, timestamp_values, bool_values,
           categorical_embed_ids, text_embed_ids,
           col_emb_table, cat_emb_table, text_batch_emb,
           W_col, b_col, W_num, b_num, W_ts, b_ts, W_cat, b_cat,
           W_text, b_text, bool_emb_table, identifier_emb, null_emb,
           mask_emb, rms_scale):
    f32 = jnp.float32
    cat_rows = _sc_cat_gather(cat_emb_table,
                              categorical_embed_ids.astype(jnp.int32).reshape(N))

    colp, textp = _project_tables(col_emb_table, text_batch_emb,
                                  W_col.astype(jnp.bfloat16),
                                  W_text.astype(jnp.bfloat16), b_col, b_text)
    col_rows, text_rows = _sc_proj_gather(
        colp, textp, column_ids.astype(jnp.int32).reshape(N),
        text_embed_ids.astype(jnp.int32).reshape(N))

    st = semantic_types.astype(f32)
    inul = is_null.astype(f32)
    itgt = is_target.astype(f32)
    a = (1.0 - itgt) * (1.0 - inul)
    bmix = (1.0 - itgt) * inul
    m = [a * (st == k).astype(f32) for k in range(6)]
    boolc = jnp.clip(bool_values.astype(f32), 0.0, NB - 1.0)
    cols = [m[0],
            m[1] * numeric_values,
            m[1]]
    cols += [m[2] * timestamp_values[..., j] for j in range(TSD)]
    cols += [m[2],
             m[3] * (boolc == 0.0).astype(f32),
             m[3] * (boolc == 1.0).astype(f32),
             m[3] * (boolc == 2.0).astype(f32),
             bmix,
             itgt,
             m[4],
             m[5],
             is_padding.astype(f32)]
    pk = jnp.stack(cols + [jnp.zeros((B, S), f32)] * (KF - len(cols)),
                   axis=-1).astype(jnp.bfloat16).reshape(N, KF)

    r_mat = jnp.concatenate(
        [identifier_emb.reshape(1, D), W_num, b_num.reshape(1, D), W_ts,
         b_ts.reshape(1, D), bool_emb_table, null_emb.reshape(1, D),
         mask_emb.reshape(1, D), b_cat.reshape(1, D),
         jnp.zeros((KF - 18, D), f32)], axis=0).astype(jnp.bfloat16)

    out = _final(cat_rows, col_rows, text_rows, pk,
                 W_cat.astype(jnp.bfloat16), r_mat, rms_scale)
    return out.reshape(B, S, D)


# speedup vs baseline: 6.9724x; 1.0533x over previous
import jax
import jax.numpy as jnp
from jax import lax
from jax.experimental import pallas as pl
from jax.experimental.pallas import tpu as pltpu
from jax.experimental.pallas import tpu_sc as plsc

B, S = 128, 200
N = B * S
D, DT = 256, 768
C, VC, U = 1000, 100000, 4096
NB, NT, TSD = 3, 7, 8
EPS = 1e-6

TOK_BLK = 512
N_BLKS = N // TOK_BLK
KF = 32

NC, NS = 2, 16
NW = NC * NS
PER_W = N // NW
CH = 80
NCH = PER_W // CH


def _pack_rows(p):
    pb = p.astype(jnp.bfloat16).astype(jnp.float32)
    bits = lax.bitcast_convert_type(pb, jnp.uint32)
    lo = bits[:, :D // 2] >> 16
    hi = bits[:, D // 2:] & jnp.uint32(0xFFFF0000)
    return lax.bitcast_convert_type(lo | hi, jnp.int32)


def _unpack_rows(w):
    u = lax.bitcast_convert_type(w, jnp.uint32)
    lo = lax.bitcast_convert_type(u << 16, jnp.float32)
    hi = lax.bitcast_convert_type(u & jnp.uint32(0xFFFF0000), jnp.float32)
    return jnp.concatenate([lo, hi], axis=1)


def _proj_body(colt_ref, textt_ref, wcol_ref, wtext_ref, bcol_ref, btext_ref,
               colp_ref, textp_ref):
    @pl.when(pl.program_id(0) == 0)
    def _():
        colp_ref[...] = _pack_rows(
            jnp.dot(colt_ref[...].astype(jnp.bfloat16),
                    wcol_ref[...].astype(jnp.bfloat16),
                    preferred_element_type=jnp.float32) + bcol_ref[...])

    textp_ref[...] = _pack_rows(
        jnp.dot(textt_ref[...].astype(jnp.bfloat16),
                wtext_ref[...].astype(jnp.bfloat16),
                preferred_element_type=jnp.float32) + btext_ref[...])


def _project_tables(col_emb_table, text_batch_emb, W_col, W_text, b_col, b_text):
    ub = U // 4
    return pl.pallas_call(
        _proj_body,
        grid=(4,),
        in_specs=[pl.BlockSpec((C, DT), lambda i: (0, 0)),
                  pl.BlockSpec((ub, DT), lambda i: (i, 0)),
                  pl.BlockSpec((DT, D), lambda i: (0, 0)),
                  pl.BlockSpec((DT, D), lambda i: (0, 0)),
                  pl.BlockSpec((1, D), lambda i: (0, 0)),
                  pl.BlockSpec((1, D), lambda i: (0, 0))],
        out_specs=(pl.BlockSpec((C, D // 2), lambda i: (0, 0)),
                   pl.BlockSpec((ub, D // 2), lambda i: (i, 0))),
        out_shape=(jax.ShapeDtypeStruct((C, D // 2), jnp.int32),
                   jax.ShapeDtypeStruct((U, D // 2), jnp.int32)),
    )(col_emb_table, text_batch_emb, W_col, W_text,
      b_col.reshape(1, D), b_text.reshape(1, D))


def _pipelined_gathers(tabs, idx_refs, outs, bufs, gsem, wsem, base):
    nt = len(tabs)

    def g(t, ci, b):
        return pltpu.make_async_copy(
            tabs[t].at[idx_refs[t].at[pl.ds(ci * CH, CH)]], bufs[t][b],
            gsem.at[t, b])

    def w(t, ci, b):
        return pltpu.make_async_copy(
            bufs[t][b], outs[t].at[pl.ds(base + ci * CH, CH)], wsem.at[t, b])

    for t in range(nt):
        g(t, 0, 0).start()
    for t in range(nt):
        g(t, 0, 0).wait()
        g(t, 1, 1).start()
        w(t, 0, 0).start()

    @pl.loop(0, (NCH - 2) // 2)
    def _(k):
        for b, delta in ((1, 1), (0, 2)):
            ci = 2 * k + delta
            for t in range(nt):
                g(t, ci, b).wait()
                w(t, ci - 1, 1 - b).wait()
                g(t, ci + 1, 1 - b).start()
                w(t, ci, b).start()

    for t in range(nt):
        g(t, NCH - 1, 1).wait()
        w(t, NCH - 2, 0).wait()
        w(t, NCH - 1, 1).start()
        w(t, NCH - 1, 1).wait()


_MESH = plsc.VectorSubcoreMesh(core_axis_name="c", subcore_axis_name="s")


def _sc_cat_body(cat_tab, cat_ids, cat_out, idx_v, buf0, buf1, gsem, wsem):
    wid = lax.axis_index("s") * NC + lax.axis_index("c")
    base = wid * PER_W
    pltpu.sync_copy(cat_ids.at[pl.ds(base, PER_W)], idx_v)
    _pipelined_gathers((cat_tab,), (idx_v,), (cat_out,), ((buf0, buf1),),
                       gsem, wsem, base)


def _sc_cat_gather(cat_tab, cat_ids):
    k = pl.kernel(
        _sc_cat_body,
        mesh=_MESH,
        out_type=jax.ShapeDtypeStruct((N, DT), jnp.float32),
        scratch_types=[pltpu.VMEM((PER_W,), jnp.int32),
                       pltpu.VMEM((CH, DT), jnp.float32),
                       pltpu.VMEM((CH, DT), jnp.float32),
                       pltpu.SemaphoreType.DMA((1, 2)),
                       pltpu.SemaphoreType.DMA((1, 2))],
    )
    return k(cat_tab, cat_ids)


def _sc_proj_body(colp_tab, textp_tab, col_ids, text_ids, col_out, text_out,
                  cidx_v, tidx_v, cbuf0, cbuf1, tbuf0, tbuf1, gsem, wsem):
    wid = lax.axis_index("s") * NC + lax.axis_index("c")
    base = wid * PER_W
    pltpu.sync_copy(col_ids.at[pl.ds(base, PER_W)], cidx_v)
    pltpu.sync_copy(text_ids.at[pl.ds(base, PER_W)], tidx_v)
    _pipelined_gathers((colp_tab, textp_tab), (cidx_v, tidx_v),
                       (col_out, text_out),
                       ((cbuf0, cbuf1), (tbuf0, tbuf1)), gsem, wsem, base)


def _sc_proj_gather(colp_tab, textp_tab, col_ids, text_ids):
    k = pl.kernel(
        _sc_proj_body,
        mesh=_MESH,
        out_type=[jax.ShapeDtypeStruct((N, D // 2), jnp.int32),
                  jax.ShapeDtypeStruct((N, D // 2), jnp.int32)],
        scratch_types=[pltpu.VMEM((PER_W,), jnp.int32),
                       pltpu.VMEM((PER_W,), jnp.int32),
                       pltpu.VMEM((CH, D // 2), jnp.int32),
                       pltpu.VMEM((CH, D // 2), jnp.int32),
                       pltpu.VMEM((CH, D // 2), jnp.int32),
                       pltpu.VMEM((CH, D // 2), jnp.int32),
                       pltpu.SemaphoreType.DMA((2, 2)),
                       pltpu.SemaphoreType.DMA((2, 2))],
    )
    return k(colp_tab, textp_tab, col_ids, text_ids)


def _final_body(cat_ref, colv_ref, textv_ref, pk_ref, wcat_ref, r_ref,
                scale_ref, out_ref):
    f32 = jnp.float32
    catd = jnp.dot(cat_ref[...].astype(jnp.bfloat16), wcat_ref[...],
                   preferred_element_type=f32)
    sel = jnp.dot(pk_ref[...], r_ref[...], preferred_element_type=f32)
    am4 = pk_ref[:, 17:18].astype(f32)
    am5 = pk_ref[:, 18:19].astype(f32)
    ipad = pk_ref[:, 19:20].astype(f32)
    x = (_unpack_rows(colv_ref[...]) + sel
         + am4 * catd + am5 * _unpack_rows(textv_ref[...]))
    xc = x - jnp.mean(x, axis=1, keepdims=True)
    h = xc * lax.rsqrt(jnp.mean(xc * xc, axis=1, keepdims=True) + EPS)
    out_ref[...] = h * scale_ref[...] * (1.0 - ipad)


def _final(cat_rows, col_rows, text_rows, pk, wcat_bf, r_mat, rms_scale):
    tok = lambda w: pl.BlockSpec((TOK_BLK, w), lambda i: (i, 0))
    rep = lambda a, b: pl.BlockSpec((a, b), lambda i: (0, 0))
    return pl.pallas_call(
        _final_body,
        grid=(N_BLKS,),
        in_specs=[tok(DT), tok(D // 2), tok(D // 2), tok(KF),
                  rep(DT, D), rep(KF, D), rep(1, D)],
        out_specs=pl.BlockSpec((TOK_BLK, D), lambda i: (i, 0)),
        out_shape=jax.ShapeDtypeStruct((N, D), jnp.float32),
    )(cat_rows, col_rows, text_rows, pk, wcat_bf, r_mat,
      rms_scale.reshape(1, D))


def kernel(semantic_types, column_ids, is_null, is_target, is_padding,
           numeric_values, timestamp_values, bool_values,
           categorical_embed_ids, text_embed_ids,
           col_emb_table, cat_emb_table, text_batch_emb,
           W_col, b_col, W_num, b_num, W_ts, b_ts, W_cat, b_cat,
           W_text, b_text, bool_emb_table, identifier_emb, null_emb,
           mask_emb, rms_scale):
    f32 = jnp.float32
    cat_rows = _sc_cat_gather(cat_emb_table,
                              categorical_embed_ids.astype(jnp.int32).reshape(N))

    colp, textp = _project_tables(col_emb_table, text_batch_emb,
                                  W_col, W_text, b_col, b_text)
    colp, textp, cat_rows = lax.optimization_barrier((colp, textp, cat_rows))
    col_rows, text_rows = _sc_proj_gather(
        colp, textp, column_ids.astype(jnp.int32).reshape(N),
        text_embed_ids.astype(jnp.int32).reshape(N))

    st = semantic_types.astype(f32)
    inul = is_null.astype(f32)
    itgt = is_target.astype(f32)
    a = (1.0 - itgt) * (1.0 - inul)
    bmix = (1.0 - itgt) * inul
    m = [a * (st == k).astype(f32) for k in range(6)]
    boolc = jnp.clip(bool_values.astype(f32), 0.0, NB - 1.0)
    cols = [m[0],
            m[1] * numeric_values,
            m[1]]
    cols += [m[2] * timestamp_values[..., j] for j in range(TSD)]
    cols += [m[2],
             m[3] * (boolc == 0.0).astype(f32),
             m[3] * (boolc == 1.0).astype(f32),
             m[3] * (boolc == 2.0).astype(f32),
             bmix,
             itgt,
             m[4],
             m[5],
             is_padding.astype(f32)]
    pk = jnp.stack(cols + [jnp.zeros((B, S), f32)] * (KF - len(cols)),
                   axis=-1).astype(jnp.bfloat16).reshape(N, KF)

    r_mat = jnp.concatenate(
        [identifier_emb.reshape(1, D), W_num, b_num.reshape(1, D), W_ts,
         b_ts.reshape(1, D), bool_emb_table, null_emb.reshape(1, D),
         mask_emb.reshape(1, D), b_cat.reshape(1, D),
         jnp.zeros((KF - 18, D), f32)], axis=0).astype(jnp.bfloat16)

    out = _final(cat_rows, col_rows, text_rows, pk,
                 W_cat.astype(jnp.bfloat16), r_mat, rms_scale)
    return out.reshape(B, S, D)
